# Initial kernel scaffold; baseline (speedup 1.0000x reference)
import jax
import jax.numpy as jnp
from jax.experimental import pallas as pl


def kernel(x, edge_index, edge_attr, batch, W1, b1, W2, b2, W3, b3, W4, b4, A, C, Wc, bc, gamma, beta, rmean, rvar):
    raise NotImplementedError("write your pallas kernel here")


# speedup vs baseline: 1.1513x; 1.1513x over previous
import functools

import jax
import jax.numpy as jnp
import numpy as np
from jax.experimental import pallas as pl
from jax.experimental.pallas import tpu as pltpu

BS, T, P = 4, 240, 22
N = BS * T * P
E = BS * T * P * (P - 1)
D4 = 152
K = 16
DC = 16
NC = 17


def _mm_body(x_ref, w_ref, b_ref, o_ref, *, relu):
    acc = jnp.dot(x_ref[...], w_ref[...], preferred_element_type=jnp.float32)
    acc = acc + b_ref[...]
    o_ref[...] = jnp.maximum(acc, 0.0) if relu else acc


def _mm(x, W, b, relu, blk=1320):
    n, din = x.shape
    dout = W.shape[1]
    return pl.pallas_call(
        functools.partial(_mm_body, relu=relu),
        grid=(n // blk,),
        in_specs=[
            pl.BlockSpec((blk, din), lambda i: (i, 0)),
            pl.BlockSpec((din, dout), lambda i: (0, 0)),
            pl.BlockSpec((1, dout), lambda i: (0, 0)),
        ],
        out_specs=pl.BlockSpec((blk, dout), lambda i: (i, 0)),
        out_shape=jax.ShapeDtypeStruct((n, dout), jnp.float32),
    )(x, W, b.reshape(1, dout))


F_BLK = 48


def _vlad_body(h_ref, a_ref, c_ref, m_ref, o_ref, acc_ref):
    h = h_ref[...]
    logits = jnp.dot(h, a_ref[...], preferred_element_type=jnp.float32)
    logits = logits - jnp.max(logits, axis=1, keepdims=True)
    expv = jnp.exp(logits)
    assign = expv / jnp.sum(expv, axis=1, keepdims=True)
    m = m_ref[...]
    asum = jnp.dot(m, assign, preferred_element_type=jnp.float32)
    c = c_ref[...]
    ssq = jnp.zeros((F_BLK, 1), jnp.float32)
    for k in range(K):
        vk = jnp.dot(m, assign[:, k:k + 1] * h, preferred_element_type=jnp.float32)
        vk = vk - asum[:, k:k + 1] * c[k:k + 1, :]
        nrm = jnp.sqrt(jnp.sum(vk * vk, axis=1, keepdims=True)) + 1e-12
        vk = vk / nrm
        ssq = ssq + jnp.sum(vk * vk, axis=1, keepdims=True)
        acc_ref[:, k * D4:(k + 1) * D4] = vk
    scale = 1.0 / (jnp.sqrt(ssq) + 1e-12)
    o_ref[...] = acc_ref[...] * scale


def _netvlad(h4, A, C):
    m = np.zeros((F_BLK, F_BLK * P), np.float32)
    for f in range(F_BLK):
        m[f, f * P:(f + 1) * P] = 1.0
    mseg = jnp.asarray(m)
    nfr = BS * T
    return pl.pallas_call(
        _vlad_body,
        grid=(nfr // F_BLK,),
        in_specs=[
            pl.BlockSpec((F_BLK * P, D4), lambda i: (i, 0)),
            pl.BlockSpec((D4, K), lambda i: (0, 0)),
            pl.BlockSpec((K, D4), lambda i: (0, 0)),
            pl.BlockSpec((F_BLK, F_BLK * P), lambda i: (0, 0)),
        ],
        out_specs=pl.BlockSpec((F_BLK, K * D4), lambda i: (i, 0)),
        out_shape=jax.ShapeDtypeStruct((nfr, K * D4), jnp.float32),
        scratch_shapes=[pltpu.VMEM((F_BLK, K * D4), jnp.float32)],
    )(h4, A, C, mseg)


def _head_body(x_ref, w_ref, bc_ref, rm_ref, rs_ref, g_ref, bt_ref, r_ref, o_ref):
    x = x_ref[0]
    acc = jnp.zeros((T, DC * NC), jnp.float32) + bc_ref[...]
    for j in range(5):
        acc = acc + jnp.dot(x[j:j + T, :], w_ref[j], preferred_element_type=jnp.float32)
    bn = (acc - rm_ref[...]) * rs_ref[...] * g_ref[...] + bt_ref[...]
    s = jax.nn.sigmoid(bn)
    q = (s - 0.5) ** 2
    red = jnp.dot(q, r_ref[...], preferred_element_type=jnp.float32) * (4.0 / DC)
    o_ref[0] = jnp.sqrt(red)


def _head(vlad, Wc, bc, gamma, beta, rmean, rvar):
    xpad = jnp.pad(vlad, ((0, 0), (2, 2), (0, 0)))
    w = jnp.transpose(Wc[:, :, :, 0], (2, 1, 0))
    rs = jax.lax.rsqrt(rvar + 1e-3)
    rmat = jnp.asarray(np.tile(np.eye(NC, dtype=np.float32), (DC, 1)))
    return pl.pallas_call(
        _head_body,
        grid=(BS,),
        in_specs=[
            pl.BlockSpec((1, T + 4, K * D4), lambda i: (i, 0, 0)),
            pl.BlockSpec((5, K * D4, DC * NC), lambda i: (0, 0, 0)),
            pl.BlockSpec((1, DC * NC), lambda i: (0, 0)),
            pl.BlockSpec((T, 1), lambda i: (0, 0)),
            pl.BlockSpec((T, 1), lambda i: (0, 0)),
            pl.BlockSpec((T, 1), lambda i: (0, 0)),
            pl.BlockSpec((T, 1), lambda i: (0, 0)),
            pl.BlockSpec((DC * NC, NC), lambda i: (0, 0)),
        ],
        out_specs=pl.BlockSpec((1, T, NC), lambda i: (i, 0, 0)),
        out_shape=jax.ShapeDtypeStruct((BS, T, NC), jnp.float32),
    )(xpad, w, bc.reshape(1, DC * NC), rmean.reshape(T, 1), rs.reshape(T, 1),
      gamma.reshape(T, 1), beta.reshape(T, 1), rmat)


def _make_agg(src, dst, w):
    deg = jnp.zeros((N,), jnp.float32).at[dst].add(w) + 1.0
    dinv = jax.lax.rsqrt(jnp.clip(deg, 1e-12, None))
    coef = w * dinv[src] * dinv[dst]
    dinv2 = dinv * dinv

    def agg(h):
        gathered = h[src] * coef[:, None]
        return (h * dinv2[:, None]).at[dst].add(gathered)

    return agg


def kernel(x, edge_index, edge_attr, batch, W1, b1, W2, b2, W3, b3, W4, b4,
           A, C, Wc, bc, gamma, beta, rmean, rvar):
    src, dst = edge_index[0], edge_index[1]
    w = edge_attr[:, 4]
    agg = _make_agg(src, dst, w)

    g1 = _mm(x, W1, b1 * 0.0, relu=False)
    h1 = jax.nn.relu(agg(g1) + b1)
    h2 = _mm(agg(h1), W2, b2, relu=True)
    h3 = _mm(agg(h2), W3, b3, relu=True)
    h4 = _mm(agg(h3), W4, b4, relu=True)

    vlad = _netvlad(h4, A, C)
    vlad = vlad.reshape(BS, T, K * D4)
    return _head(vlad, Wc, bc, gamma, beta, rmean, rvar)


# speedup vs baseline: 3.8474x; 3.3417x over previous
import functools

import jax
import jax.numpy as jnp
import numpy as np
from jax import lax
from jax.experimental import pallas as pl
from jax.experimental.pallas import tpu as pltpu
from jax.experimental.pallas import tpu_sc as plsc

BS, T, P = 4, 240, 22
N = BS * T * P
E = BS * T * P * (P - 1)
D4 = 152
K = 16
DC = 16
NC = 17

NTILES = 32
CHUNK = 128
NCH = 112
E_PAD = NTILES * NCH * CHUNK
EROWS = E_PAD // CHUNK
ROWS_RD = 1328
NPAD = 16 * ROWS_RD


DEG_ROWS = EROWS // 16

_SC_PARAMS = pltpu.CompilerParams(
    needs_layout_passes=False, use_tc_tiling_on_sc=False)


def _sc_mesh():
    return plsc.VectorSubcoreMesh(core_axis_name="c", subcore_axis_name="s")


def _wid():
    return lax.axis_index("s") * 2 + lax.axis_index("c")


def _newton_rsqrt(x):
    xi = plsc.bitcast(x, jnp.int32)
    y = plsc.bitcast(jnp.int32(0x5F3759DF) - (xi >> 1), jnp.float32)
    for _ in range(4):
        y = y * (1.5 - 0.5 * x * y * y)
    return y


def _scale_scatter(cf_b, rows, accum, dst_i):
    iota16 = lax.iota(jnp.int32, 16)
    zero16 = iota16 ^ iota16

    def scale(e, carry):
        ev = lax.broadcast_in_dim(e, (16,), ())
        cvec = plsc.load_gather(cf_b, [zero16, ev])
        rowv = plsc.load_gather(rows, [ev, iota16])
        plsc.store_scatter(rows, [ev, iota16], rowv * cvec)
        return carry

    lax.fori_loop(0, CHUNK, scale, 0)
    pltpu.sync_copy(rows, accum.at[dst_i.at[0]], add=True)


def _sc_prep(srcp, dstp, wp, g1, zeros1, zeros16):
    def body(srcp_h, dstp_h, wp_h, g1_h, z1_h, z16_h,
             coef_o, dinv_o, out0, out1,
             src_i, dst_i, w_b, cf_b, rows, degv, zb1, zv, rb, dacc, accum, sem):
        c = lax.axis_index("c")
        s = lax.axis_index("s")
        r0 = s * ROWS_RD
        pltpu.sync_copy(z1_h.at[pl.ds(r0, ROWS_RD)], zb1)
        pltpu.sync_copy(zb1, dacc.at[pl.ds(r0, ROWS_RD)])
        pltpu.sync_copy(z16_h.at[pl.ds(r0, ROWS_RD)], zv)
        pltpu.sync_copy(zv, accum.at[pl.ds(r0, ROWS_RD)])
        plsc.subcore_barrier()

        def dchunk(i, carry):
            row = s * DEG_ROWS + i
            pltpu.sync_copy(dstp_h.at[pl.ds(row, 1)], dst_i)
            pltpu.sync_copy(wp_h.at[pl.ds(row, 1)], w_b)
            pltpu.sync_copy(w_b.at[0], dacc.at[dst_i.at[0]], add=True)
            return carry

        lax.fori_loop(0, DEG_ROWS, dchunk, 0)
        plsc.subcore_barrier()

        pltpu.sync_copy(dacc, degv)

        def dl(i, carry):
            x = degv[pl.ds(i * 16, 16)] + 1.0
            degv[pl.ds(i * 16, 16)] = _newton_rsqrt(x)
            return carry

        lax.fori_loop(0, NPAD // 16, dl, 0)

        @pl.when(c == 0)
        def _():
            pltpu.sync_copy(degv.at[pl.ds(r0, ROWS_RD)], dinv_o.at[pl.ds(r0, ROWS_RD)])

        base = _wid() * NCH

        def chunk(i, carry):
            row = base + i
            pltpu.sync_copy(srcp_h.at[pl.ds(row, 1)], src_i)
            pltpu.sync_copy(dstp_h.at[pl.ds(row, 1)], dst_i)
            pltpu.sync_copy(wp_h.at[pl.ds(row, 1)], w_b)
            for k in range(CHUNK // 16):
                sv = src_i[0, pl.ds(k * 16, 16)]
                dv = dst_i[0, pl.ds(k * 16, 16)]
                wv = w_b[0, pl.ds(k * 16, 16)]
                cf = wv * plsc.load_gather(degv, [sv]) * plsc.load_gather(degv, [dv])
                cf_b[0, pl.ds(k * 16, 16)] = cf
            pltpu.sync_copy(cf_b, coef_o.at[pl.ds(row, 1)])
            pltpu.async_copy(g1_h.at[src_i.at[0]], rows, sem).wait()
            _scale_scatter(cf_b, rows, accum, dst_i)
            return carry

        lax.fori_loop(0, NCH, chunk, 0)
        plsc.subcore_barrier()

        pltpu.sync_copy(accum.at[pl.ds(r0, ROWS_RD)], rb)

        @pl.when(c == 0)
        def _():
            pltpu.sync_copy(rb, out0.at[pl.ds(r0, ROWS_RD)])

        @pl.when(c == 1)
        def _():
            pltpu.sync_copy(rb, out1.at[pl.ds(r0, ROWS_RD)])

    f = pl.kernel(
        body,
        out_type=(jax.ShapeDtypeStruct((EROWS, CHUNK), jnp.float32),
                  jax.ShapeDtypeStruct((NPAD,), jnp.float32),
                  jax.ShapeDtypeStruct((NPAD, 16), jnp.float32),
                  jax.ShapeDtypeStruct((NPAD, 16), jnp.float32)),
        mesh=_sc_mesh(),
        compiler_params=_SC_PARAMS,
        scratch_types=[
            pltpu.VMEM((1, CHUNK), jnp.int32),
            pltpu.VMEM((1, CHUNK), jnp.int32),
            pltpu.VMEM((1, CHUNK), jnp.float32),
            pltpu.VMEM((1, CHUNK), jnp.float32),
            pltpu.VMEM((CHUNK, 16), jnp.float32),
            pltpu.VMEM((NPAD,), jnp.float32),
            pltpu.VMEM((ROWS_RD,), jnp.float32),
            pltpu.VMEM((ROWS_RD, 16), jnp.float32),
            pltpu.VMEM((ROWS_RD, 16), jnp.float32),
            pltpu.VMEM_SHARED((NPAD,), jnp.float32),
            pltpu.VMEM_SHARED((NPAD, 16), jnp.float32),
            pltpu.SemaphoreType.DMA,
        ],
    )
    return f(srcp, dstp, wp, g1, zeros1, zeros16)


def _sc_agg(hparts, srcp, dstp, coefp, zeros16):
    nparts = len(hparts)

    def body(*refs):
        hs = refs[0:nparts]
        srcp_h, dstp_h, coefp_h, z16_h = refs[nparts:nparts + 4]
        outs0 = refs[nparts + 4: nparts + 4 + nparts]
        outs1 = refs[nparts + 4 + nparts: nparts + 4 + 2 * nparts]
        src_i, dst_i, cf_b, rows, zv, rb, accum, sem = refs[nparts + 4 + 2 * nparts:]
        c = lax.axis_index("c")
        s = lax.axis_index("s")
        r0 = s * ROWS_RD
        pltpu.sync_copy(z16_h.at[pl.ds(r0, ROWS_RD)], zv)
        pltpu.sync_copy(zv, accum.at[pl.ds(r0, ROWS_RD)])
        base = _wid() * NCH
        for p in range(nparts):
            plsc.subcore_barrier()

            def chunk(i, carry, _h=hs[p]):
                row = base + i
                pltpu.sync_copy(srcp_h.at[pl.ds(row, 1)], src_i)
                pltpu.sync_copy(dstp_h.at[pl.ds(row, 1)], dst_i)
                pltpu.sync_copy(coefp_h.at[pl.ds(row, 1)], cf_b)
                pltpu.async_copy(_h.at[src_i.at[0]], rows, sem).wait()
                _scale_scatter(cf_b, rows, accum, dst_i)
                return carry

            lax.fori_loop(0, NCH, chunk, 0)
            plsc.subcore_barrier()
            pltpu.sync_copy(accum.at[pl.ds(r0, ROWS_RD)], rb)
            pltpu.sync_copy(zv, accum.at[pl.ds(r0, ROWS_RD)])

            @pl.when(c == 0)
            def _(p=p):
                pltpu.sync_copy(rb, outs0[p].at[pl.ds(r0, ROWS_RD)])

            @pl.when(c == 1)
            def _(p=p):
                pltpu.sync_copy(rb, outs1[p].at[pl.ds(r0, ROWS_RD)])

    sds = jax.ShapeDtypeStruct((NPAD, 16), jnp.float32)
    f = pl.kernel(
        body,
        out_type=tuple([sds] * (2 * nparts)),
        mesh=_sc_mesh(),
        compiler_params=_SC_PARAMS,
        scratch_types=[
            pltpu.VMEM((1, CHUNK), jnp.int32),
            pltpu.VMEM((1, CHUNK), jnp.int32),
            pltpu.VMEM((1, CHUNK), jnp.float32),
            pltpu.VMEM((CHUNK, 16), jnp.float32),
            pltpu.VMEM((ROWS_RD, 16), jnp.float32),
            pltpu.VMEM((ROWS_RD, 16), jnp.float32),
            pltpu.VMEM_SHARED((NPAD, 16), jnp.float32),
            pltpu.SemaphoreType.DMA,
        ],
    )
    outs = f(*hparts, srcp, dstp, coefp, zeros16)
    return outs[:nparts], outs[nparts:]


MM_BLK = 1320


def _mm_body(x_ref, w_ref, b_ref, o_ref, *, relu):
    acc = jnp.dot(x_ref[...], w_ref[...], preferred_element_type=jnp.float32)
    acc = acc + b_ref[...]
    o_ref[...] = jnp.maximum(acc, 0.0) if relu else acc


def _mm(x, W, b, relu):
    n, din = x.shape
    dout = W.shape[1]
    return pl.pallas_call(
        functools.partial(_mm_body, relu=relu),
        grid=(n // MM_BLK,),
        in_specs=[
            pl.BlockSpec((MM_BLK, din), lambda i: (i, 0)),
            pl.BlockSpec((din, dout), lambda i: (0, 0)),
            pl.BlockSpec((1, dout), lambda i: (0, 0)),
        ],
        out_specs=pl.BlockSpec((MM_BLK, dout), lambda i: (i, 0)),
        out_shape=jax.ShapeDtypeStruct((n, dout), jnp.float32),
    )(x, W, b.reshape(1, dout))


def _l1_body(p0_ref, p1_ref, h_ref, s_ref, b_ref, o_ref):
    d2 = s_ref[...] * s_ref[...]
    t = p0_ref[...] + p1_ref[...] + d2 * h_ref[...] + b_ref[...]
    o_ref[...] = jnp.maximum(t, 0.0)


def _l1(p0, p1, h, dinv2col, b):
    d = h.shape[1]
    return pl.pallas_call(
        _l1_body,
        grid=(N // MM_BLK,),
        in_specs=[
            pl.BlockSpec((MM_BLK, d), lambda i: (i, 0)),
            pl.BlockSpec((MM_BLK, d), lambda i: (i, 0)),
            pl.BlockSpec((MM_BLK, d), lambda i: (i, 0)),
            pl.BlockSpec((MM_BLK, 1), lambda i: (i, 0)),
            pl.BlockSpec((1, d), lambda i: (0, 0)),
        ],
        out_specs=pl.BlockSpec((MM_BLK, d), lambda i: (i, 0)),
        out_shape=jax.ShapeDtypeStruct((N, d), jnp.float32),
    )(p0, p1, h, dinv2col, b.reshape(1, d))


def _mm_agg_body(p0_ref, p1_ref, h_ref, s_ref, w_ref, b_ref, o_ref):
    t = p0_ref[...] + p1_ref[...] + s_ref[...] * s_ref[...] * h_ref[...]
    acc = jnp.dot(t, w_ref[...], preferred_element_type=jnp.float32) + b_ref[...]
    o_ref[...] = jnp.maximum(acc, 0.0)


def _mm_agg(p0, p1, h, dinv2col, W, b):
    din = h.shape[1]
    dout = W.shape[1]
    return pl.pallas_call(
        _mm_agg_body,
        grid=(N // MM_BLK,),
        in_specs=[
            pl.BlockSpec((MM_BLK, din), lambda i: (i, 0)),
            pl.BlockSpec((MM_BLK, din), lambda i: (i, 0)),
            pl.BlockSpec((MM_BLK, din), lambda i: (i, 0)),
            pl.BlockSpec((MM_BLK, 1), lambda i: (i, 0)),
            pl.BlockSpec((din, dout), lambda i: (0, 0)),
            pl.BlockSpec((1, dout), lambda i: (0, 0)),
        ],
        out_specs=pl.BlockSpec((MM_BLK, dout), lambda i: (i, 0)),
        out_shape=jax.ShapeDtypeStruct((N, dout), jnp.float32),
    )(p0, p1, h, dinv2col, W, b.reshape(1, dout))


F_BLK = 48


def _vlad_body(h_ref, a_ref, c_ref, m_ref, o_ref, acc_ref):
    h = h_ref[...]
    logits = jnp.dot(h, a_ref[...], preferred_element_type=jnp.float32)
    logits = logits - jnp.max(logits, axis=1, keepdims=True)
    expv = jnp.exp(logits)
    assign = expv / jnp.sum(expv, axis=1, keepdims=True)
    m = m_ref[...]
    asum = jnp.dot(m, assign, preferred_element_type=jnp.float32)
    c = c_ref[...]
    ssq = jnp.zeros((F_BLK, 1), jnp.float32)
    for k in range(K):
        vk = jnp.dot(m, assign[:, k:k + 1] * h, preferred_element_type=jnp.float32)
        vk = vk - asum[:, k:k + 1] * c[k:k + 1, :]
        nrm = jnp.sqrt(jnp.sum(vk * vk, axis=1, keepdims=True)) + 1e-12
        vk = vk / nrm
        ssq = ssq + jnp.sum(vk * vk, axis=1, keepdims=True)
        acc_ref[:, k * D4:(k + 1) * D4] = vk
    scale = 1.0 / (jnp.sqrt(ssq) + 1e-12)
    o_ref[...] = acc_ref[...] * scale


def _netvlad(h4, A, C):
    m = np.zeros((F_BLK, F_BLK * P), np.float32)
    for f in range(F_BLK):
        m[f, f * P:(f + 1) * P] = 1.0
    mseg = jnp.asarray(m)
    nfr = BS * T
    return pl.pallas_call(
        _vlad_body,
        grid=(nfr // F_BLK,),
        in_specs=[
            pl.BlockSpec((F_BLK * P, D4), lambda i: (i, 0)),
            pl.BlockSpec((D4, K), lambda i: (0, 0)),
            pl.BlockSpec((K, D4), lambda i: (0, 0)),
            pl.BlockSpec((F_BLK, F_BLK * P), lambda i: (0, 0)),
        ],
        out_specs=pl.BlockSpec((F_BLK, K * D4), lambda i: (i, 0)),
        out_shape=jax.ShapeDtypeStruct((nfr, K * D4), jnp.float32),
        scratch_shapes=[pltpu.VMEM((F_BLK, K * D4), jnp.float32)],
    )(h4, A, C, mseg)


def _head_body(x_ref, w_ref, bc_ref, rm_ref, rs_ref, g_ref, bt_ref, r_ref, o_ref):
    x = x_ref[0]
    acc = jnp.zeros((T, DC * NC), jnp.float32) + bc_ref[...]
    for j in range(5):
        acc = acc + jnp.dot(x[j:j + T, :], w_ref[j], preferred_element_type=jnp.float32)
    bn = (acc - rm_ref[...]) * rs_ref[...] * g_ref[...] + bt_ref[...]
    s = jax.nn.sigmoid(bn)
    q = (s - 0.5) ** 2
    red = jnp.dot(q, r_ref[...], preferred_element_type=jnp.float32) * (4.0 / DC)
    o_ref[0] = jnp.sqrt(red)


def _head(vlad, Wc, bc, gamma, beta, rmean, rvar):
    xpad = jnp.pad(vlad, ((0, 0), (2, 2), (0, 0)))
    w = jnp.transpose(Wc[:, :, :, 0], (2, 1, 0))
    rs = lax.rsqrt(rvar + 1e-3)
    rmat = jnp.asarray(np.tile(np.eye(NC, dtype=np.float32), (DC, 1)))
    return pl.pallas_call(
        _head_body,
        grid=(BS,),
        in_specs=[
            pl.BlockSpec((1, T + 4, K * D4), lambda i: (i, 0, 0)),
            pl.BlockSpec((5, K * D4, DC * NC), lambda i: (0, 0, 0)),
            pl.BlockSpec((1, DC * NC), lambda i: (0, 0)),
            pl.BlockSpec((T, 1), lambda i: (0, 0)),
            pl.BlockSpec((T, 1), lambda i: (0, 0)),
            pl.BlockSpec((T, 1), lambda i: (0, 0)),
            pl.BlockSpec((T, 1), lambda i: (0, 0)),
            pl.BlockSpec((DC * NC, NC), lambda i: (0, 0)),
        ],
        out_specs=pl.BlockSpec((1, T, NC), lambda i: (i, 0, 0)),
        out_shape=jax.ShapeDtypeStruct((BS, T, NC), jnp.float32),
    )(xpad, w, bc.reshape(1, DC * NC), rmean.reshape(T, 1), rs.reshape(T, 1),
      gamma.reshape(T, 1), beta.reshape(T, 1), rmat)


def kernel(x, edge_index, edge_attr, batch, W1, b1, W2, b2, W3, b3, W4, b4,
           A, C, Wc, bc, gamma, beta, rmean, rvar):
    src = edge_index[0]
    dst = edge_index[1]
    w = edge_attr[:, 4]

    pad = E_PAD - E
    srcp = jnp.concatenate([src, jnp.zeros((pad,), jnp.int32)]).reshape(EROWS, CHUNK)
    dstp = jnp.concatenate([dst, jnp.full((pad,), N, jnp.int32)]).reshape(EROWS, CHUNK)
    wp = jnp.concatenate([w, jnp.zeros((pad,), jnp.float32)]).reshape(EROWS, CHUNK)

    zeros1 = jnp.zeros((NPAD,), jnp.float32)
    zeros16 = jnp.zeros((NPAD, 16), jnp.float32)

    g1 = _mm(x, W1, jnp.zeros((16,), jnp.float32), relu=False)

    coefp, dinv, a10, a11 = _sc_prep(srcp, dstp, wp, g1, zeros1, zeros16)
    dinvcol = dinv[:N, None]

    h1 = _l1(a10[:N], a11[:N], g1, dinvcol, b1)

    (p0,), (p1,) = _sc_agg([h1], srcp, dstp, coefp, zeros16)
    h2 = _mm_agg(p0[:N], p1[:N], h1, dinvcol, W2, b2)

    ps0, ps1 = _sc_agg([h2[:, :16], h2[:, 16:]], srcp, dstp, coefp, zeros16)
    p0 = jnp.concatenate([o[:N] for o in ps0], axis=1)
    p1 = jnp.concatenate([o[:N] for o in ps1], axis=1)
    h3 = _mm_agg(p0, p1, h2, dinvcol, W3, b3)

    ps0, ps1 = _sc_agg([h3[:, 16 * j:16 * (j + 1)] for j in range(4)],
                       srcp, dstp, coefp, zeros16)
    p0 = jnp.concatenate([o[:N] for o in ps0], axis=1)
    p1 = jnp.concatenate([o[:N] for o in ps1], axis=1)
    h4 = _mm_agg(p0, p1, h3, dinvcol, W4, b4)

    vlad = _netvlad(h4, A, C)
    vlad = vlad.reshape(BS, T, K * D4)
    return _head(vlad, Wc, bc, gamma, beta, rmean, rvar)
